# Initial kernel scaffold; baseline (speedup 1.0000x reference)
import jax
import jax.numpy as jnp
from jax.experimental import pallas as pl


def kernel(tokens, memory_r, memory_v, memory_o, conn_r, conn_v, conn_o):
    raise NotImplementedError("write your pallas kernel here")


# speedup vs baseline: 5.4323x; 5.4323x over previous
import functools

import jax
import jax.numpy as jnp
from jax import lax
from jax.experimental import pallas as pl
from jax.experimental.pallas import tpu as pltpu
from jax.experimental.pallas import tpu_sc as plsc

S = 1024
TOKEN_BITS = 32
H = 4
P = 11
RP = 2 * P
VB = 10
OB = 12
L = 16
NW = 32
TPH = NW // H
QPT = S // TPH
WKEYS = 16
NWIN = S // WKEYS
NG = QPT // L


def _bcast(ref, i):
    return plsc.load_gather(ref, [jnp.full((L,), i, jnp.int32)])


def _ctz16(m):
    low = m & (-m)
    f = low.astype(jnp.float32)
    return (lax.bitcast_convert_type(f, jnp.int32) >> 23) - 127


def _k1_body(memr, aqh, ak, vlo, vhi, connv, memv,
             firstj, projw,
             ak_v, aqh_v, idx_v, vals_v, fj_v, mx_v, vlo_v, vhi_v, connv_v,
             memv_v, pw_v, sem):
    wid = lax.axis_index("s") * 2 + lax.axis_index("c")
    h = wid // TPH
    part = wid % TPH
    q0 = h * S + part * QPT

    pltpu.sync_copy(ak.at[pl.ds(h * S, S)], ak_v)
    pltpu.sync_copy(aqh.at[pl.ds(q0, QPT)], aqh_v)

    for g in range(NG):
        fj_v[pl.ds(g * L, L)] = jnp.full((L,), S, jnp.int32)

    def scan_window(w):
        aqg = [aqh_v[pl.ds(g * L, L)] for g in range(NG)]
        for j in range(WKEYS):
            akj = _bcast(ak_v, w * WKEYS + j)
            for g in range(NG):
                idx_v[pl.ds(j * QPT + g * L, L)] = aqg[g] + akj
        copies = [
            pltpu.async_copy(memr.at[idx_v.at[pl.ds(j * QPT, QPT)]],
                             vals_v.at[pl.ds(j * QPT, QPT)], sem)
            for j in range(WKEYS)
        ]
        for cp in copies:
            cp.wait()
        mx = jnp.zeros((L,), jnp.int32)
        for g in range(NG):
            fj = fj_v[pl.ds(g * L, L)]
            m = jnp.zeros((L,), jnp.int32)
            for j in range(WKEYS):
                v = vals_v[pl.ds(j * QPT + g * L, L)]
                m = m | jnp.where(v > 0.5, jnp.int32(1 << j), jnp.int32(0))
            upd = (fj >= S) & (m != 0)
            fj = jnp.where(upd, w * WKEYS + _ctz16(m), fj)
            fj_v[pl.ds(g * L, L)] = fj
            mx = jnp.maximum(mx, fj)
        mx_v[pl.ds(0, L)] = mx

    def win_body(w, mx):
        @pl.when(mx >= S)
        def _():
            scan_window(w)

        return jnp.max(mx_v[pl.ds(0, L)])

    lax.fori_loop(0, NWIN, win_body, jnp.int32(S))
    pltpu.sync_copy(fj_v, firstj.at[pl.ds(q0, QPT)])

    j0 = part * QPT
    pltpu.sync_copy(vlo.at[pl.ds(j0, QPT)], vlo_v)
    pltpu.sync_copy(vhi.at[pl.ds(j0, QPT)], vhi_v)
    pltpu.sync_copy(connv.at[pl.ds(h * TOKEN_BITS * L, TOKEN_BITS * L)],
                    connv_v)
    pltpu.sync_copy(memv.at[h], memv_v)

    for g in range(NG):
        pw_v[pl.ds(g * L, L)] = jnp.zeros((L,), jnp.int32)

    def t_body(t, carry):
        cbs = [_bcast(connv_v, t * L + b) for b in range(VB)]
        bit_t = jnp.full((L,), 1, jnp.int32) << t
        tful = jnp.full((L,), t, jnp.int32)
        for g in range(NG):
            lo = vlo_v[pl.ds(g * L, L)]
            hi = vhi_v[pl.ds(g * L, L)]
            acc = jnp.zeros((L,), jnp.int32)
            for b in range(VB):
                cb = cbs[b]
                blo = (lo >> (cb & 31)) & 1
                bhi = (hi >> ((cb - TOKEN_BITS) & 31)) & 1
                bit = jnp.where(cb < TOKEN_BITS, blo, bhi)
                acc = acc | (bit << b)
            pv = plsc.load_gather(memv_v, [tful, acc])
            word = pw_v[pl.ds(g * L, L)]
            pw_v[pl.ds(g * L, L)] = word | jnp.where(pv > 0.5, bit_t,
                                                     jnp.int32(0))
        return carry

    lax.fori_loop(0, TOKEN_BITS, t_body, jnp.int32(0))
    pltpu.sync_copy(pw_v, projw.at[pl.ds(q0, QPT)])


def _k2_body(firstj, projw, conno, memo, outt,
             fj_all, pw_all, conno_v, memo_v, out_v):
    t = lax.axis_index("s") * 2 + lax.axis_index("c")
    pltpu.sync_copy(firstj, fj_all)
    pltpu.sync_copy(projw, pw_all)
    pltpu.sync_copy(conno, conno_v)
    pltpu.sync_copy(memo.at[t], memo_v)

    cbs = [_bcast(conno_v, t * L + b) for b in range(OB)]
    hbs = [cb >> 5 for cb in cbs]
    bps = [cb & 31 for cb in cbs]

    def g_body(g, carry):
        ws = []
        for h in range(H):
            fj = fj_all[pl.ds(h * S + g * L, L)]
            ex = fj < S
            idx = jnp.where(ex, fj, jnp.int32(0))
            w = plsc.load_gather(pw_all, [jnp.full((L,), h, jnp.int32), idx])
            ws.append(jnp.where(ex, w, jnp.int32(0)))
        acc = jnp.zeros((L,), jnp.int32)
        for b in range(OB):
            hb = hbs[b]
            w01 = jnp.where(hb < 1, ws[0], ws[1])
            w23 = jnp.where(hb < 3, ws[2], ws[3])
            wsel = jnp.where(hb < 2, w01, w23)
            acc = acc | (((wsel >> bps[b]) & 1) << b)
        out_v[pl.ds(g * L, L)] = plsc.load_gather(memo_v, [acc])
        return carry

    lax.fori_loop(0, S // L, g_body, jnp.int32(0))
    pltpu.sync_copy(out_v, outt.at[pl.ds(t * S, S)])


@jax.jit
def kernel(tokens, memory_r, memory_v, memory_o, conn_r, conn_v, conn_o):
    pos = jnp.arange(S)
    shifts = jnp.arange(P - 1, -1, -1)
    pb = ((pos[:, None] >> shifts[None, :]) & 1).astype(jnp.int32)
    wr = (jnp.int32(1) << jnp.arange(RP, dtype=jnp.int32))

    is_q = conn_r < P
    qg = jnp.take(pb, jnp.where(is_q, conn_r, 0), axis=1)
    kg = jnp.take(pb, jnp.where(is_q, 0, conn_r - P), axis=1)
    aq = jnp.sum(qg * (is_q.astype(jnp.int32) * wr)[None], axis=2)
    ak = jnp.sum(kg * ((1 - is_q.astype(jnp.int32)) * wr)[None], axis=2)
    hoff = (jnp.arange(H, dtype=jnp.int32) << RP)[:, None]
    aqh = (aq.T.astype(jnp.int32) + hoff).reshape(-1)
    ak = ak.T.reshape(-1).astype(jnp.int32)

    wtok = jnp.int32(1) << jnp.arange(TOKEN_BITS, dtype=jnp.int32)
    vlo = jnp.sum(tokens * wtok[None, :], axis=1).astype(jnp.int32)
    vhi = jnp.sum(pb * (jnp.int32(1) << jnp.arange(P, dtype=jnp.int32))[None, :],
                  axis=1).astype(jnp.int32)

    connv = jnp.pad(conn_v, ((0, 0), (0, 0), (0, L - VB))).reshape(-1)
    conno = jnp.pad(conn_o, ((0, 0), (0, L - OB))).reshape(-1)
    memr = memory_r.reshape(-1)

    mesh = plsc.VectorSubcoreMesh(core_axis_name="c", subcore_axis_name="s")

    cparams = pltpu.CompilerParams(needs_layout_passes=False)

    k1 = functools.partial(
        pl.kernel, mesh=mesh, compiler_params=cparams,
        out_type=(jax.ShapeDtypeStruct((H * S,), jnp.int32),
                  jax.ShapeDtypeStruct((H * S,), jnp.int32)),
        scratch_types=[
            pltpu.VMEM((S,), jnp.int32),
            pltpu.VMEM((QPT,), jnp.int32),
            pltpu.VMEM((WKEYS * QPT,), jnp.int32),
            pltpu.VMEM((WKEYS * QPT,), jnp.float32),
            pltpu.VMEM((QPT,), jnp.int32),
            pltpu.VMEM((L,), jnp.int32),
            pltpu.VMEM((QPT,), jnp.int32),
            pltpu.VMEM((QPT,), jnp.int32),
            pltpu.VMEM((TOKEN_BITS * L,), jnp.int32),
            pltpu.VMEM((TOKEN_BITS, 1 << VB), jnp.float32),
            pltpu.VMEM((QPT,), jnp.int32),
            pltpu.SemaphoreType.DMA,
        ],
    )(_k1_body)
    firstj, projw = k1(memr, aqh, ak, vlo, vhi, connv, memory_v)

    k2 = functools.partial(
        pl.kernel, mesh=mesh, compiler_params=cparams,
        out_type=jax.ShapeDtypeStruct((TOKEN_BITS * S,), jnp.float32),
        scratch_types=[
            pltpu.VMEM((H * S,), jnp.int32),
            pltpu.VMEM((H, S), jnp.int32),
            pltpu.VMEM((TOKEN_BITS * L,), jnp.int32),
            pltpu.VMEM((1 << OB,), jnp.float32),
            pltpu.VMEM((S,), jnp.float32),
        ],
    )(_k2_body)
    outt = k2(firstj, projw.reshape(H, S), conno, memory_o)

    return outt.reshape(TOKEN_BITS, S).T


# speedup vs baseline: 5.6000x; 1.0309x over previous
import functools

import jax
import jax.numpy as jnp
from jax import lax
from jax.experimental import pallas as pl
from jax.experimental.pallas import tpu as pltpu
from jax.experimental.pallas import tpu_sc as plsc

S = 1024
TOKEN_BITS = 32
H = 4
P = 11
RP = 2 * P
VB = 10
OB = 12
L = 16
NW = 32
TPH = NW // H
QPT = S // TPH
WKEYS = 16
NWIN = S // WKEYS
NG = QPT // L


def _bcast(ref, i):
    return plsc.load_gather(ref, [jnp.full((L,), i, jnp.int32)])


def _ctz16(m):
    low = m & (-m)
    f = low.astype(jnp.float32)
    return (lax.bitcast_convert_type(f, jnp.int32) >> 23) - 127


def _k1_body(memr, aqh, ak, vlo, vhi, connv, memv,
             firstj, projw,
             ak_v, aqh_v, idx_v, vals_v, fj_v, mx_v, vlo_v, vhi_v, connv_v,
             memv_v, pw_v, sem, sem2):
    wid = lax.axis_index("s") * 2 + lax.axis_index("c")
    h = wid // TPH
    part = wid % TPH
    q0 = h * S + part * QPT
    j0 = part * QPT

    stage = [
        pltpu.async_copy(vlo.at[pl.ds(j0, QPT)], vlo_v, sem2),
        pltpu.async_copy(vhi.at[pl.ds(j0, QPT)], vhi_v, sem2),
        pltpu.async_copy(
            connv.at[pl.ds(h * TOKEN_BITS * L, TOKEN_BITS * L)], connv_v,
            sem2),
        pltpu.async_copy(memv.at[h], memv_v, sem2),
    ]

    pltpu.sync_copy(ak.at[pl.ds(h * S, S)], ak_v)
    pltpu.sync_copy(aqh.at[pl.ds(q0, QPT)], aqh_v)

    for g in range(NG):
        fj_v[pl.ds(g * L, L)] = jnp.full((L,), S, jnp.int32)

    def scan_window(w):
        aqg = [aqh_v[pl.ds(g * L, L)] for g in range(NG)]
        for j in range(WKEYS):
            akj = _bcast(ak_v, w * WKEYS + j)
            for g in range(NG):
                idx_v[pl.ds(j * QPT + g * L, L)] = aqg[g] + akj
        copies = [
            pltpu.async_copy(memr.at[idx_v.at[pl.ds(j * QPT, QPT)]],
                             vals_v.at[pl.ds(j * QPT, QPT)], sem)
            for j in range(WKEYS)
        ]
        for cp in copies:
            cp.wait()
        mx = jnp.zeros((L,), jnp.int32)
        for g in range(NG):
            fj = fj_v[pl.ds(g * L, L)]
            m = jnp.zeros((L,), jnp.int32)
            for j in range(WKEYS):
                v = vals_v[pl.ds(j * QPT + g * L, L)]
                m = m | jnp.where(v > 0.5, jnp.int32(1 << j), jnp.int32(0))
            upd = (fj >= S) & (m != 0)
            fj = jnp.where(upd, w * WKEYS + _ctz16(m), fj)
            fj_v[pl.ds(g * L, L)] = fj
            mx = jnp.maximum(mx, fj)
        mx_v[pl.ds(0, L)] = mx

    def win_body(w, mx):
        @pl.when(mx >= S)
        def _():
            scan_window(w)

        return jnp.max(mx_v[pl.ds(0, L)])

    lax.fori_loop(0, NWIN, win_body, jnp.int32(S))
    pltpu.sync_copy(fj_v, firstj.at[pl.ds(q0, QPT)])

    for cp in stage:
        cp.wait()

    for g in range(NG):
        pw_v[pl.ds(g * L, L)] = jnp.zeros((L,), jnp.int32)

    def t_body(t, carry):
        cbs = [_bcast(connv_v, t * L + b) for b in range(VB)]
        mbs = [cb < TOKEN_BITS for cb in cbs]
        sbs = [jnp.where(m, cb, cb - TOKEN_BITS)
               for cb, m in zip(cbs, mbs)]
        bit_t = jnp.full((L,), 1, jnp.int32) << t
        tful = jnp.full((L,), t, jnp.int32)
        for g in range(NG):
            lo = vlo_v[pl.ds(g * L, L)]
            hi = vhi_v[pl.ds(g * L, L)]
            acc = jnp.zeros((L,), jnp.int32)
            for b in range(VB):
                src = jnp.where(mbs[b], lo, hi)
                acc = acc | (((src >> sbs[b]) & 1) << b)
            pv = plsc.load_gather(memv_v, [tful, acc])
            word = pw_v[pl.ds(g * L, L)]
            pw_v[pl.ds(g * L, L)] = word | jnp.where(pv > 0.5, bit_t,
                                                     jnp.int32(0))
        return carry

    lax.fori_loop(0, TOKEN_BITS, t_body, jnp.int32(0))
    pltpu.sync_copy(pw_v, projw.at[pl.ds(q0, QPT)])


def _k2_body(firstj, projw, conno, memo, outt,
             fj_all, pw_all, conno_v, memo_v, out_v):
    t = lax.axis_index("s") * 2 + lax.axis_index("c")
    pltpu.sync_copy(firstj, fj_all)
    pltpu.sync_copy(projw, pw_all)
    pltpu.sync_copy(conno, conno_v)
    pltpu.sync_copy(memo.at[t], memo_v)

    cbs = [_bcast(conno_v, t * L + b) for b in range(OB)]
    hbs = [cb >> 5 for cb in cbs]
    bps = [cb & 31 for cb in cbs]

    def g_body(g, carry):
        ws = []
        for h in range(H):
            fj = fj_all[pl.ds(h * S + g * L, L)]
            ex = fj < S
            idx = jnp.where(ex, fj, jnp.int32(0))
            w = plsc.load_gather(pw_all, [jnp.full((L,), h, jnp.int32), idx])
            ws.append(jnp.where(ex, w, jnp.int32(0)))
        acc = jnp.zeros((L,), jnp.int32)
        for b in range(OB):
            hb = hbs[b]
            w01 = jnp.where(hb < 1, ws[0], ws[1])
            w23 = jnp.where(hb < 3, ws[2], ws[3])
            wsel = jnp.where(hb < 2, w01, w23)
            acc = acc | (((wsel >> bps[b]) & 1) << b)
        out_v[pl.ds(g * L, L)] = plsc.load_gather(memo_v, [acc])
        return carry

    lax.fori_loop(0, S // L, g_body, jnp.int32(0))
    pltpu.sync_copy(out_v, outt.at[pl.ds(t * S, S)])


@jax.jit
def kernel(tokens, memory_r, memory_v, memory_o, conn_r, conn_v, conn_o):
    pos = jnp.arange(S)
    shifts = jnp.arange(P - 1, -1, -1)
    pb = ((pos[:, None] >> shifts[None, :]) & 1).astype(jnp.int32)
    wr = (jnp.int32(1) << jnp.arange(RP, dtype=jnp.int32))

    is_q = conn_r < P
    qg = jnp.take(pb, jnp.where(is_q, conn_r, 0), axis=1)
    kg = jnp.take(pb, jnp.where(is_q, 0, conn_r - P), axis=1)
    aq = jnp.sum(qg * (is_q.astype(jnp.int32) * wr)[None], axis=2)
    ak = jnp.sum(kg * ((1 - is_q.astype(jnp.int32)) * wr)[None], axis=2)
    hoff = (jnp.arange(H, dtype=jnp.int32) << RP)[:, None]
    aqh = (aq.T.astype(jnp.int32) + hoff).reshape(-1)
    ak = ak.T.reshape(-1).astype(jnp.int32)

    wtok = jnp.int32(1) << jnp.arange(TOKEN_BITS, dtype=jnp.int32)
    vlo = jnp.sum(tokens * wtok[None, :], axis=1).astype(jnp.int32)
    vhi = jnp.sum(pb * (jnp.int32(1) << jnp.arange(P, dtype=jnp.int32))[None, :],
                  axis=1).astype(jnp.int32)

    connv = jnp.pad(conn_v, ((0, 0), (0, 0), (0, L - VB))).reshape(-1)
    conno = jnp.pad(conn_o, ((0, 0), (0, L - OB))).reshape(-1)
    memr = memory_r.reshape(-1)

    mesh = plsc.VectorSubcoreMesh(core_axis_name="c", subcore_axis_name="s")

    cparams = pltpu.CompilerParams(needs_layout_passes=False)

    k1 = functools.partial(
        pl.kernel, mesh=mesh, compiler_params=cparams,
        out_type=(jax.ShapeDtypeStruct((H * S,), jnp.int32),
                  jax.ShapeDtypeStruct((H * S,), jnp.int32)),
        scratch_types=[
            pltpu.VMEM((S,), jnp.int32),
            pltpu.VMEM((QPT,), jnp.int32),
            pltpu.VMEM((WKEYS * QPT,), jnp.int32),
            pltpu.VMEM((WKEYS * QPT,), jnp.float32),
            pltpu.VMEM((QPT,), jnp.int32),
            pltpu.VMEM((L,), jnp.int32),
            pltpu.VMEM((QPT,), jnp.int32),
            pltpu.VMEM((QPT,), jnp.int32),
            pltpu.VMEM((TOKEN_BITS * L,), jnp.int32),
            pltpu.VMEM((TOKEN_BITS, 1 << VB), jnp.float32),
            pltpu.VMEM((QPT,), jnp.int32),
            pltpu.SemaphoreType.DMA,
            pltpu.SemaphoreType.DMA,
        ],
    )(_k1_body)
    firstj, projw = k1(memr, aqh, ak, vlo, vhi, connv, memory_v)

    k2 = functools.partial(
        pl.kernel, mesh=mesh, compiler_params=cparams,
        out_type=jax.ShapeDtypeStruct((TOKEN_BITS * S,), jnp.float32),
        scratch_types=[
            pltpu.VMEM((H * S,), jnp.int32),
            pltpu.VMEM((H, S), jnp.int32),
            pltpu.VMEM((TOKEN_BITS * L,), jnp.int32),
            pltpu.VMEM((1 << OB,), jnp.float32),
            pltpu.VMEM((S,), jnp.float32),
        ],
    )(_k2_body)
    outt = k2(firstj, projw.reshape(H, S), conno, memory_o)

    return outt.reshape(TOKEN_BITS, S).T
